# baseline (speedup 1.0000x reference)
import functools

import jax
import jax.numpy as jnp
from jax import lax
from jax.experimental import pallas as pl
from jax.experimental.pallas import tpu as pltpu
from jax.experimental.pallas import tpu_sc as plsc

B, T, D = 4, 8192, 2048
K = 8
DK = D // K
NW = 32
LANES = 16


def _make_sc_gather():
    mesh = plsc.VectorSubcoreMesh(core_axis_name="c", subcore_axis_name="s")

    @functools.partial(
        pl.kernel,
        mesh=mesh,
        out_type=jax.ShapeDtypeStruct((B * D,), jnp.float32),
        scratch_types=[
            pltpu.VMEM((LANES,), jnp.int32),
            pltpu.VMEM((LANES,), jnp.int32),
            pltpu.VMEM((1, DK), jnp.float32),
            pltpu.SemaphoreType.DMA,
        ],
        compiler_params=pltpu.CompilerParams(needs_layout_passes=False),
    )
    def sc_gather(feat_hbm, len_hbm, out_hbm, len_v, idx_v, row_v, sem):
        wid = lax.axis_index("s") * 2 + lax.axis_index("c")
        pltpu.sync_copy(len_hbm, len_v)
        lane = lax.iota(jnp.int32, LANES)
        widv = lax.broadcast_in_dim(wid, (LANES,), ())
        lg = lax.rem(widv + lane, lax.broadcast_in_dim(jnp.int32(NW), (LANES,), ()))
        b = lax.shift_right_logical(lg, 3)
        kk = lg - b * 8
        lens = plsc.load_gather(len_v, [b])
        idx_v[...] = (b * T + lens - 1) * 8 + kk
        pltpu.async_copy(feat_hbm.at[idx_v.at[pl.ds(0, 1)]], row_v, sem).wait()
        pltpu.sync_copy(row_v.at[0], out_hbm.at[pl.ds(wid * DK, DK)])

    return sc_gather


_sc_gather = _make_sc_gather()


def kernel(features, _mask, lengths):
    feat = features.reshape(B * T * K, DK)
    lengths_p = jnp.pad(lengths, (0, LANES - B))
    out = _sc_gather(feat, lengths_p)
    return out.reshape(B, 1, D)


# speedup vs baseline: 16.2611x; 16.2611x over previous
import functools

import jax
import jax.numpy as jnp
from jax.experimental import pallas as pl
from jax.experimental.pallas import tpu as pltpu
from jax.experimental.pallas import tpu_sc as plsc

B, T, D = 4, 8192, 2048


def _make_sc_gather():
    mesh = plsc.ScalarSubcoreMesh(axis_name="c", num_cores=1)

    @functools.partial(
        pl.kernel,
        mesh=mesh,
        out_type=jax.ShapeDtypeStruct((B * D,), jnp.float32),
        scratch_types=[
            pltpu.SMEM((B,), jnp.int32),
            pltpu.SemaphoreType.DMA,
        ],
        compiler_params=pltpu.CompilerParams(needs_layout_passes=False),
    )
    def sc_gather(feat_hbm, len_hbm, out_hbm, len_s, sem):
        pltpu.sync_copy(len_hbm, len_s)
        for b in range(B):
            row = b * T + len_s[b] - 1
            pltpu.async_copy(feat_hbm.at[row], out_hbm.at[pl.ds(b * D, D)], sem)
        pltpu.make_async_copy(out_hbm, out_hbm, sem).wait()

    return sc_gather


_sc_gather = _make_sc_gather()


def kernel(features, _mask, lengths):
    feat = features.reshape(B * T, D)
    out = _sc_gather(feat, lengths)
    return out.reshape(B, 1, D)
